# baseline (speedup 1.0000x reference)
import functools

import jax
import jax.numpy as jnp
from jax import lax
from jax.experimental import pallas as pl
from jax.experimental.pallas import tpu as pltpu
from jax.experimental.pallas import tpu_sc as plsc

EMB_D = 64
BATCH = 16384
NC = 2
NS = 16
NW = NC * NS
B_W = BATCH // NW
ICH = 128
NIC = B_W // ICH


def _mf_kernel(ids1_hbm, ids2_hbm, e1t_hbm, e2t_hbm, out_hbm,
               idx1_v, idx2_v, g1_v, g2_v, acc_v, sem1, sem2):
    wid = lax.axis_index("s") * NC + lax.axis_index("c")

    pltpu.sync_copy(ids1_hbm.at[wid], idx1_v)
    pltpu.sync_copy(ids2_hbm.at[wid], idx2_v)

    def issue(d, b):
        for c in range(NIC):
            pltpu.async_copy(
                e1t_hbm.at[d].at[idx1_v.at[pl.ds(c * ICH, ICH)]],
                g1_v.at[b, pl.ds(c * ICH, ICH)], sem1)
            pltpu.async_copy(
                e2t_hbm.at[d].at[idx2_v.at[pl.ds(c * ICH, ICH)]],
                g2_v.at[b, pl.ds(c * ICH, ICH)], sem2)

    def drain(b):
        for c in range(NIC):
            pltpu.make_async_copy(
                e1t_hbm.at[0].at[idx1_v.at[pl.ds(c * ICH, ICH)]],
                g1_v.at[b, pl.ds(c * ICH, ICH)], sem1).wait()
            pltpu.make_async_copy(
                e2t_hbm.at[0].at[idx2_v.at[pl.ds(c * ICH, ICH)]],
                g2_v.at[b, pl.ds(c * ICH, ICH)], sem2).wait()

    def zinit(k, _):
        acc_v[pl.ds(k * 16, 16)] = jnp.zeros((16,), jnp.float32)
        return 0

    lax.fori_loop(0, B_W // 16, zinit, 0)

    issue(0, 0)
    issue(1, 1)

    def step(d, _):
        b = lax.rem(d, 2)
        drain(b)

        @pl.when(d + 2 < EMB_D)
        def _():
            issue(d + 2, b)

        def mac(k, _):
            s = pl.ds(k * 16, 16)
            acc_v[s] = acc_v[s] + g1_v[b, s] * g2_v[b, s]
            return 0

        lax.fori_loop(0, B_W // 16, mac, 0)
        return 0

    lax.fori_loop(0, EMB_D, step, 0)

    pltpu.sync_copy(acc_v, out_hbm.at[wid])


@jax.jit
def kernel(ids1, ids2, emb1, emb2):
    mesh = plsc.VectorSubcoreMesh(core_axis_name="c", subcore_axis_name="s",
                                  num_cores=NC, num_subcores=NS)
    k = functools.partial(
        pl.kernel,
        out_type=jax.ShapeDtypeStruct((NW, B_W), jnp.float32),
        mesh=mesh,
        compiler_params=pltpu.CompilerParams(use_tc_tiling_on_sc=False),
        scratch_types=[
            pltpu.VMEM((B_W,), jnp.int32),
            pltpu.VMEM((B_W,), jnp.int32),
            pltpu.VMEM((2, B_W), jnp.float32),
            pltpu.VMEM((2, B_W), jnp.float32),
            pltpu.VMEM((B_W,), jnp.float32),
            pltpu.SemaphoreType.DMA,
            pltpu.SemaphoreType.DMA,
        ],
    )(_mf_kernel)
    ids1_2d = ids1.astype(jnp.int32).reshape(NW, B_W)
    ids2_2d = ids2.astype(jnp.int32).reshape(NW, B_W)
    out = k(ids1_2d, ids2_2d, emb1.T, emb2.T)
    return out.reshape(BATCH, 1)
